# baseline (device time: 28083 ns/iter reference)
import jax
import jax.numpy as jnp
from jax import lax
from jax.experimental import pallas as pl
from jax.experimental.pallas import tpu as pltpu

N_DEV = 16
CHUNK = 16


def kernel(x, Wq, Wo, K_ext, V_ext):
    B, Sq, D = x.shape
    H_loc = Wq.shape[1]
    Dh = K_ext.shape[-1]
    H = H_loc // Dh
    Dout = Wo.shape[1]

    def body(x_ref, wq_ref, wo_ref, k_ref, v_ref, out_ref,
             part_ref, stage_ref, rs_recv, ag_send, ag_recv,
             rs_send_sems, rs_recv_sems, ag_send_sems, ag_recv_sems):
        my = lax.axis_index("i")

        wq = wq_ref[...].astype(jnp.bfloat16)
        for b in range(B):
            xb = x_ref[b].astype(jnp.bfloat16)
            q = jnp.dot(xb, wq, preferred_element_type=jnp.float32)
            acc = jnp.zeros((Sq, Dout), jnp.float32)
            for h in range(H):
                qh = (q[:, h * Dh:(h + 1) * Dh] * 0.125).astype(jnp.bfloat16)
                kh = k_ref[b, :, h, :].astype(jnp.bfloat16)
                vh = v_ref[b, :, h, :].astype(jnp.bfloat16)
                s = jnp.dot(qh, kh.T, preferred_element_type=jnp.float32)
                m = jnp.max(s, axis=-1, keepdims=True)
                p = jnp.exp(s - m)
                l = jnp.sum(p, axis=-1, keepdims=True)
                o = jnp.dot(p.astype(jnp.bfloat16), vh,
                            preferred_element_type=jnp.float32) / l
                woh = wo_ref[h * Dh:(h + 1) * Dh, :].astype(jnp.bfloat16)
                acc = acc + jnp.dot(o.astype(jnp.bfloat16), woh,
                                    preferred_element_type=jnp.float32)
            part_ref[b * (N_DEV // B):(b + 1) * (N_DEV // B)] = (
                acc.reshape(N_DEV // B, CHUNK, Dout))
            stage_ref[b * (N_DEV // B):(b + 1) * (N_DEV // B)] = (
                acc.astype(jnp.bfloat16).reshape(N_DEV // B, CHUNK, Dout))

        barrier = pltpu.get_barrier_semaphore()
        for o in range(1, N_DEV):
            pl.semaphore_signal(barrier, inc=1, device_id=(my ^ o,),
                                device_id_type=pl.DeviceIdType.MESH)
        pl.semaphore_wait(barrier, N_DEV - 1)

        rs = []
        for o in range(1, N_DEV):
            peer = my ^ o
            rdma = pltpu.make_async_remote_copy(
                src_ref=stage_ref.at[peer],
                dst_ref=rs_recv.at[o],
                send_sem=rs_send_sems.at[o],
                recv_sem=rs_recv_sems.at[o],
                device_id=(peer,),
                device_id_type=pl.DeviceIdType.MESH,
            )
            rdma.start()
            rs.append(rdma)

        red = part_ref[my]
        for o in range(1, N_DEV):
            rs[o - 1].wait_recv()
            red = red + rs_recv[o].astype(jnp.float32)

        ag_send[...] = red.astype(jnp.bfloat16)
        ag = []
        for o in range(1, N_DEV):
            rdma = pltpu.make_async_remote_copy(
                src_ref=ag_send,
                dst_ref=ag_recv.at[o],
                send_sem=ag_send_sems.at[o],
                recv_sem=ag_recv_sems.at[o],
                device_id=(my ^ o,),
                device_id_type=pl.DeviceIdType.MESH,
            )
            rdma.start()
            ag.append(rdma)

        out_ref[my] = red
        for o in range(1, N_DEV):
            ag[o - 1].wait_recv()
            out_ref[my ^ o] = ag_recv[o].astype(jnp.float32)

        for o in range(1, N_DEV):
            rs[o - 1].wait_send()
            ag[o - 1].wait_send()

    out = pl.pallas_call(
        body,
        out_shape=jax.ShapeDtypeStruct((N_DEV, CHUNK, Dout), jnp.float32),
        in_specs=[pl.BlockSpec(memory_space=pltpu.VMEM)] * 5,
        out_specs=pl.BlockSpec(memory_space=pltpu.VMEM),
        scratch_shapes=[
            pltpu.VMEM((N_DEV, CHUNK, Dout), jnp.float32),
            pltpu.VMEM((N_DEV, CHUNK, Dout), jnp.bfloat16),
            pltpu.VMEM((N_DEV, CHUNK, Dout), jnp.bfloat16),
            pltpu.VMEM((CHUNK, Dout), jnp.bfloat16),
            pltpu.VMEM((N_DEV, CHUNK, Dout), jnp.bfloat16),
            pltpu.SemaphoreType.DMA((N_DEV,)),
            pltpu.SemaphoreType.DMA((N_DEV,)),
            pltpu.SemaphoreType.DMA((N_DEV,)),
            pltpu.SemaphoreType.DMA((N_DEV,)),
        ],
        compiler_params=pltpu.CompilerParams(collective_id=0),
    )(x, Wq, Wo, K_ext, V_ext)
    return out.reshape(B, Sq, Dout)


# device time: 23675 ns/iter; 1.1862x vs baseline; 1.1862x over previous
import jax
import jax.numpy as jnp
from jax import lax
from jax.experimental import pallas as pl
from jax.experimental.pallas import tpu as pltpu

N_DEV = 16
CHUNK = 16


def kernel(x, Wq, Wo, K_ext, V_ext):
    B, Sq, D = x.shape
    H_loc = Wq.shape[1]
    Dh = K_ext.shape[-1]
    H = H_loc // Dh
    Dout = Wo.shape[1]
    CPB = N_DEV // B

    def body(x_ref, wq_ref, wo_ref, k_ref, v_ref, out_ref,
             part_ref, stage_ref, obuf_ref, rs_recv, ag_send, ag_recv,
             rs_send_sems, rs_recv_sems, ag_send_sems, ag_recv_sems):
        my = lax.axis_index("i")

        barrier = pltpu.get_barrier_semaphore()
        for o in range(1, N_DEV):
            pl.semaphore_signal(barrier, inc=1, device_id=(my ^ o,),
                                device_id_type=pl.DeviceIdType.MESH)

        wq = wq_ref[...].astype(jnp.bfloat16)
        wo = wo_ref[...].astype(jnp.bfloat16)
        x2d = x_ref[...].reshape(B * Sq, D).astype(jnp.bfloat16)
        q = jnp.dot(x2d, wq, preferred_element_type=jnp.float32)

        rs = []
        for o in range(1, N_DEV):
            peer = my ^ o
            rs.append(pltpu.make_async_remote_copy(
                src_ref=stage_ref.at[peer],
                dst_ref=rs_recv.at[my],
                send_sem=rs_send_sems.at[o],
                recv_sem=rs_recv_sems.at[my],
                device_id=(peer,),
                device_id_type=pl.DeviceIdType.MESH,
            ))
        for b in range(B):
            for h in range(H):
                qh = (q[b * Sq:(b + 1) * Sq, h * Dh:(h + 1) * Dh]
                      * 0.125).astype(jnp.bfloat16)
                kh = k_ref[b, :, h, :].astype(jnp.bfloat16)
                vh = v_ref[b, :, h, :].astype(jnp.bfloat16)
                s = jnp.dot(qh, kh.T, preferred_element_type=jnp.float32)
                m = jnp.max(s, axis=-1, keepdims=True)
                p = jnp.exp(s - m)
                l = jnp.sum(p, axis=-1, keepdims=True)
                o_h = jnp.dot(p.astype(jnp.bfloat16), vh,
                              preferred_element_type=jnp.float32) / l
                obuf_ref[b * Sq:(b + 1) * Sq,
                         h * Dh:(h + 1) * Dh] = o_h.astype(jnp.bfloat16)
            accb = jnp.dot(obuf_ref[b * Sq:(b + 1) * Sq, :], wo,
                           preferred_element_type=jnp.float32)
            part_ref[b * CPB:(b + 1) * CPB] = accb.reshape(CPB, CHUNK, Dout)
            stage_ref[b * CPB:(b + 1) * CPB] = (
                accb.astype(jnp.bfloat16).reshape(CPB, CHUNK, Dout))

            if b == 0:
                pl.semaphore_wait(barrier, N_DEV - 1)
            for o in range(1, N_DEV):
                peer = my ^ o
                rdma = rs[o - 1]

                @pl.when((peer // CPB) == b)
                def _(rdma=rdma):
                    rdma.start()

        rs_recv[my] = stage_ref[my]
        for o in range(1, N_DEV):
            sender = my ^ o
            pltpu.make_async_remote_copy(
                src_ref=stage_ref.at[sender],
                dst_ref=rs_recv.at[sender],
                send_sem=rs_send_sems.at[o],
                recv_sem=rs_recv_sems.at[sender],
                device_id=(sender,),
                device_id_type=pl.DeviceIdType.MESH,
            ).wait_recv()
        red = jnp.sum(rs_recv[...].astype(jnp.float32), axis=0)

        ag_send[...] = red.astype(jnp.bfloat16)
        ag = []
        for o in range(1, N_DEV):
            rdma = pltpu.make_async_remote_copy(
                src_ref=ag_send,
                dst_ref=ag_recv.at[my],
                send_sem=ag_send_sems.at[o],
                recv_sem=ag_recv_sems.at[my],
                device_id=(my ^ o,),
                device_id_type=pl.DeviceIdType.MESH,
            )
            rdma.start()
            ag.append(rdma)

        ag_recv[my] = ag_send[...]
        for o in range(1, N_DEV):
            sender = my ^ o
            pltpu.make_async_remote_copy(
                src_ref=ag_send,
                dst_ref=ag_recv.at[sender],
                send_sem=ag_send_sems.at[o],
                recv_sem=ag_recv_sems.at[sender],
                device_id=(sender,),
                device_id_type=pl.DeviceIdType.MESH,
            ).wait_recv()
        out_ref[...] = ag_recv[...].astype(jnp.float32)

        for o in range(1, N_DEV):
            rs[o - 1].wait_send()
            ag[o - 1].wait_send()

    out = pl.pallas_call(
        body,
        out_shape=jax.ShapeDtypeStruct((N_DEV, CHUNK, Dout), jnp.float32),
        in_specs=[pl.BlockSpec(memory_space=pltpu.VMEM)] * 5,
        out_specs=pl.BlockSpec(memory_space=pltpu.VMEM),
        scratch_shapes=[
            pltpu.VMEM((N_DEV, CHUNK, Dout), jnp.float32),
            pltpu.VMEM((N_DEV, CHUNK, Dout), jnp.bfloat16),
            pltpu.VMEM((B * Sq, H * Dh), jnp.bfloat16),
            pltpu.VMEM((N_DEV, CHUNK, Dout), jnp.bfloat16),
            pltpu.VMEM((CHUNK, Dout), jnp.bfloat16),
            pltpu.VMEM((N_DEV, CHUNK, Dout), jnp.bfloat16),
            pltpu.SemaphoreType.DMA((N_DEV,)),
            pltpu.SemaphoreType.DMA((N_DEV,)),
            pltpu.SemaphoreType.DMA((N_DEV,)),
            pltpu.SemaphoreType.DMA((N_DEV,)),
        ],
        compiler_params=pltpu.CompilerParams(collective_id=0),
    )(x, Wq, Wo, K_ext, V_ext)
    return out.reshape(B, Sq, Dout)


# device time: 22141 ns/iter; 1.2684x vs baseline; 1.0693x over previous
import jax
import jax.numpy as jnp
from jax import lax
from jax.experimental import pallas as pl
from jax.experimental.pallas import tpu as pltpu

N_DEV = 16
NH = 2
CROWS = 8


def kernel(x, Wq, Wo, K_ext, V_ext):
    B, Sq, D = x.shape
    H_loc = Wq.shape[1]
    Dh = K_ext.shape[-1]
    H = H_loc // Dh
    Dout = Wo.shape[1]

    def body(x_ref, wq_ref, wo_ref, k_ref, v_ref, out_ref,
             stage_ref, obuf_ref, rs_recv, ag_send, ag_recv,
             rs_send_sems, rs_recv_sems, ag_send_sems, ag_recv_sems):
        my = lax.axis_index("i")

        barrier = pltpu.get_barrier_semaphore()
        for o in range(1, N_DEV):
            pl.semaphore_signal(barrier, inc=1, device_id=(my ^ o,),
                                device_id_type=pl.DeviceIdType.MESH)

        wq = (wq_ref[...] * 0.125).astype(jnp.bfloat16)
        wo = wo_ref[...].astype(jnp.bfloat16)
        x2d = x_ref[...].reshape(B * Sq, D).astype(jnp.bfloat16)
        q = jnp.dot(x2d, wq, preferred_element_type=jnp.float32)

        def rs_rdma(half, o):
            peer = my ^ o
            return pltpu.make_async_remote_copy(
                src_ref=stage_ref.at[half, peer],
                dst_ref=rs_recv.at[half, my],
                send_sem=rs_send_sems.at[half, o],
                recv_sem=rs_recv_sems.at[half, my],
                device_id=(peer,),
                device_id_type=pl.DeviceIdType.MESH,
            )

        def rs_wait(half, o):
            sender = my ^ o
            pltpu.make_async_remote_copy(
                src_ref=stage_ref.at[half, sender],
                dst_ref=rs_recv.at[half, sender],
                send_sem=rs_send_sems.at[half, o],
                recv_sem=rs_recv_sems.at[half, sender],
                device_id=(sender,),
                device_id_type=pl.DeviceIdType.MESH,
            ).wait_recv()

        def ag_rdma(half, o):
            return pltpu.make_async_remote_copy(
                src_ref=ag_send.at[half],
                dst_ref=ag_recv.at[half, my],
                send_sem=ag_send_sems.at[half, o],
                recv_sem=ag_recv_sems.at[half, my],
                device_id=(my ^ o,),
                device_id_type=pl.DeviceIdType.MESH,
            )

        def ag_wait(half, o):
            sender = my ^ o
            pltpu.make_async_remote_copy(
                src_ref=ag_send.at[half],
                dst_ref=ag_recv.at[half, sender],
                send_sem=ag_send_sems.at[half, o],
                recv_sem=ag_recv_sems.at[half, sender],
                device_id=(sender,),
                device_id_type=pl.DeviceIdType.MESH,
            ).wait_recv()

        rs_started = []
        ag_started = []

        def compute_half(b):
            for h in range(H):
                qh = q[b * Sq:(b + 1) * Sq,
                       h * Dh:(h + 1) * Dh].astype(jnp.bfloat16)
                kh = k_ref[b, :, h, :].astype(jnp.bfloat16)
                vh = v_ref[b, :, h, :].astype(jnp.bfloat16)
                s = jnp.dot(qh, kh.T, preferred_element_type=jnp.float32)
                m = jnp.max(s, axis=-1, keepdims=True)
                p = jnp.exp(s - m)
                l = jnp.sum(p, axis=-1, keepdims=True)
                o_h = jnp.dot(p.astype(jnp.bfloat16), vh,
                              preferred_element_type=jnp.float32) / l
                obuf_ref[b * Sq:(b + 1) * Sq,
                         h * Dh:(h + 1) * Dh] = o_h.astype(jnp.bfloat16)
            accb = jnp.dot(obuf_ref[b * Sq:(b + 1) * Sq, :], wo,
                           preferred_element_type=jnp.float32)
            stage_ref[b] = accb.astype(jnp.bfloat16).reshape(
                N_DEV, CROWS, Dout)

        def reduce_and_ag(half):
            rs_recv[half, my] = stage_ref[half, my]
            for o in range(1, N_DEV):
                rs_wait(half, o)
            red = jnp.sum(rs_recv[half].astype(jnp.float32), axis=0)
            ag_send[half] = red.astype(jnp.bfloat16)
            for o in range(1, N_DEV):
                r = ag_rdma(half, o)
                r.start()
                ag_started.append(r)
            ag_recv[half, my] = ag_send[half]

        compute_half(0)
        pl.semaphore_wait(barrier, N_DEV - 1)
        for o in range(1, N_DEV):
            r = rs_rdma(0, o)
            r.start()
            rs_started.append(r)

        compute_half(1)
        for o in range(1, N_DEV):
            r = rs_rdma(1, o)
            r.start()
            rs_started.append(r)

        reduce_and_ag(0)
        reduce_and_ag(1)

        for o in range(1, N_DEV):
            ag_wait(0, o)
            ag_wait(1, o)
        out_ref[...] = ag_recv[...].astype(jnp.float32)

        for r in rs_started:
            r.wait_send()
        for r in ag_started:
            r.wait_send()

    out = pl.pallas_call(
        body,
        out_shape=jax.ShapeDtypeStruct((NH, N_DEV, CROWS, Dout), jnp.float32),
        in_specs=[pl.BlockSpec(memory_space=pltpu.VMEM)] * 5,
        out_specs=pl.BlockSpec(memory_space=pltpu.VMEM),
        scratch_shapes=[
            pltpu.VMEM((NH, N_DEV, CROWS, Dout), jnp.bfloat16),
            pltpu.VMEM((B * Sq, H * Dh), jnp.bfloat16),
            pltpu.VMEM((NH, N_DEV, CROWS, Dout), jnp.bfloat16),
            pltpu.VMEM((NH, CROWS, Dout), jnp.bfloat16),
            pltpu.VMEM((NH, N_DEV, CROWS, Dout), jnp.bfloat16),
            pltpu.SemaphoreType.DMA((NH, N_DEV)),
            pltpu.SemaphoreType.DMA((NH, N_DEV)),
            pltpu.SemaphoreType.DMA((NH, N_DEV)),
            pltpu.SemaphoreType.DMA((NH, N_DEV)),
        ],
        compiler_params=pltpu.CompilerParams(collective_id=0),
    )(x, Wq, Wo, K_ext, V_ext)
    return out.reshape(B, Sq, Dout)
